# Initial kernel scaffold; baseline (speedup 1.0000x reference)
import jax
import jax.numpy as jnp
from jax.experimental import pallas as pl


def kernel(x1, edge_index1, edge_attr1, x2, edge_index2, edge_attr2, W1, b1, W2, b2, Wl1, bl1, Wl2, bl2):
    raise NotImplementedError("write your pallas kernel here")


# speedup vs baseline: 49.6061x; 49.6061x over previous
import functools

import jax
import jax.numpy as jnp
from jax import lax
from jax.experimental import pallas as pl
from jax.experimental.pallas import tpu as pltpu
from jax.experimental.pallas import tpu_sc as plsc

N = 10000
E = 320000
D = 128
H = 8

NC = 2
NS = 16
L = 16

NP = 10240
NB = NP // NS
EP = 327680
BLK = 128
NBLK = EP // BLK
BPT = NBLK // NS
CBLK = 32
NCHUNK = BPT // CBLK


def _leaky(x):
    return jnp.where(x >= 0, x, 0.01 * x)


def _mm_body(x_ref, w_ref, o_ref):
    o_ref[...] = jnp.dot(x_ref[...], w_ref[...],
                         preferred_element_type=jnp.float32)


def _vdot(a, b):
    acc = a[:, 0:1] * b[0:1, :]
    for k in range(1, a.shape[1]):
        acc = acc + a[:, k:k + 1] * b[k:k + 1, :]
    return acc


def _readout_body(ph_ref, w2_ref, b2_ref, g_ref, h2_scr):
    for gi in range(NC):
        t = jnp.dot(ph_ref[gi], w2_ref[...],
                    preferred_element_type=jnp.float32) + b2_ref[...]
        h2_scr[...] = _leaky(t)

        def _acc(i, acc):
            return acc + h2_scr[pl.ds(i * 8, 8), :]
        acc = lax.fori_loop(0, N // 8, _acc,
                            jnp.zeros((8, H), jnp.float32))
        a4 = acc[0:4, :] + acc[4:8, :]
        a2 = a4[0:2, :] + a4[2:4, :]
        g_ref[gi:gi + 1, :] = a2[0:1, :] + a2[1:2, :]


def _mlp_body(g_ref, wl1_ref, bl1_ref, wl2_ref, bl2_ref, o_ref):
    z = jnp.concatenate([g_ref[0:1, :], g_ref[1:2, :]], axis=1)
    r = _leaky(_vdot(z, wl1_ref[...]) + bl1_ref[...])
    o_ref[...] = _vdot(r, wl2_ref[...]) + bl2_ref[...]


def _sc_body(rowH, colH, ewH, yH, zH, b1H, phH,
             row2d, col2d, ewf, urows, ul2d, sl2d, degl, dinvl, b1v,
             deg_sp, u_sp, s_sp, gsem, ssem):
    c = lax.axis_index("c")
    s = lax.axis_index("s")
    nb = s * NB
    bb0 = s * BPT

    iota = lax.iota(jnp.int32, L)
    io3 = jnp.right_shift(iota, 3)
    io_f = jnp.bitwise_and(iota, 7)
    e16m = [16 * i + iota for i in range(BLK // L)]
    f16v = [jnp.full((L,), f, dtype=jnp.int32) for f in range(H)]
    zero16 = jnp.zeros((L,), jnp.float32)

    pltpu.sync_copy(b1H, b1v)

    def _z_deg(i, _):
        degl[pl.ds(i * L, L)] = zero16
        return 0
    lax.fori_loop(0, NB // L, _z_deg, 0)
    pltpu.sync_copy(degl, deg_sp.at[pl.ds(nb, NB)])
    pltpu.sync_copy(zH.at[pl.ds(nb, NB)], s_sp.at[pl.ds(nb, NB)])
    plsc.subcore_barrier()

    def _deg_chunk(k, _):
        bb = bb0 + CBLK * k
        pltpu.sync_copy(colH.at[c].at[pl.ds(bb, CBLK)], col2d)
        pltpu.sync_copy(ewH.at[c].at[pl.ds(bb * BLK, CBLK * BLK)], ewf)

        def _deg_blk(j, _):
            pltpu.async_copy(ewf.at[pl.ds(j * BLK, BLK)],
                             deg_sp.at[col2d.at[j]], ssem, add=True)
            return 0
        lax.fori_loop(0, CBLK, _deg_blk, 0)

        def _deg_drain(j, _):
            pltpu.make_async_copy(ewf.at[pl.ds(0, BLK)],
                                  deg_sp.at[pl.ds(0, BLK)], ssem).wait()
            return 0
        lax.fori_loop(0, CBLK, _deg_drain, 0)
        return 0
    lax.fori_loop(0, NCHUNK, _deg_chunk, 0)
    plsc.subcore_barrier()

    pltpu.sync_copy(deg_sp.at[pl.ds(nb, NB)], degl)

    def _dinv(i, _):
        d = degl[pl.ds(i * L, L)] + 1.0
        half = 0.5 * d
        y = 1.0 / d

        def _newton(_, y):
            return y * (1.5 - half * y * y)
        y = lax.fori_loop(0, 22, _newton, y)
        dinvl[pl.ds(i * L, L)] = y
        return 0
    lax.fori_loop(0, NB // L, _dinv, 0)

    pltpu.sync_copy(yH.at[c].at[pl.ds(nb, NB)], ul2d)

    def _scale_u(i, _):
        r16 = io3 + 2 * i
        u = plsc.load_gather(ul2d, [r16, io_f])
        dv = plsc.load_gather(dinvl, [r16])
        plsc.store_scatter(ul2d, [r16, io_f], u * dv)
        return 0
    lax.fori_loop(0, (NB * H) // L, _scale_u, 0)
    pltpu.sync_copy(ul2d, u_sp.at[pl.ds(nb, NB)])
    plsc.subcore_barrier()

    def _propagate():
        def _chunk(k, _):
            bb = bb0 + CBLK * k
            pltpu.sync_copy(rowH.at[c].at[pl.ds(bb, CBLK)], row2d)
            pltpu.sync_copy(colH.at[c].at[pl.ds(bb, CBLK)], col2d)
            pltpu.sync_copy(ewH.at[c].at[pl.ds(bb * BLK, CBLK * BLK)], ewf)

            def _gfire(j, _):
                pltpu.async_copy(u_sp.at[row2d.at[j]], urows.at[j], gsem)
                return 0
            lax.fori_loop(0, CBLK, _gfire, 0)

            def _gdrain(j, _):
                pltpu.make_async_copy(u_sp.at[pl.ds(0, BLK)],
                                      urows.at[0], gsem).wait()
                return 0
            lax.fori_loop(0, CBLK, _gdrain, 0)

            def _mul_blk(j, _):
                j16 = jnp.full((L,), j, dtype=jnp.int32)
                for i in range(BLK // L):
                    ew16 = ewf[pl.ds(j * BLK + 16 * i, L)]
                    for f in range(H):
                        idx = [j16, e16m[i], f16v[f]]
                        uv = plsc.load_gather(urows, idx)
                        plsc.store_scatter(urows, idx, uv * ew16)
                pltpu.async_copy(urows.at[j], s_sp.at[col2d.at[j]],
                                 ssem, add=True)
                return 0
            lax.fori_loop(0, CBLK, _mul_blk, 0)

            def _sdrain(j, _):
                pltpu.make_async_copy(urows.at[0],
                                      s_sp.at[pl.ds(0, BLK)], ssem).wait()
                return 0
            lax.fori_loop(0, CBLK, _sdrain, 0)
            return 0
        lax.fori_loop(0, NCHUNK, _chunk, 0)

    _propagate()
    plsc.subcore_barrier()

    pltpu.sync_copy(s_sp.at[pl.ds(nb, NB)], sl2d)
    b1vec = b1v[...]

    def _inter(i, _):
        r16 = io3 + 2 * i
        sv = plsc.load_gather(sl2d, [r16, io_f])
        uv = plsc.load_gather(ul2d, [r16, io_f])
        dv = plsc.load_gather(dinvl, [r16])
        z = dv * (sv + uv) + b1vec
        h = jnp.where(z >= 0, z, 0.01 * z)
        plsc.store_scatter(ul2d, [r16, io_f], dv * h)
        return 0
    lax.fori_loop(0, (NB * H) // L, _inter, 0)
    pltpu.sync_copy(zH.at[pl.ds(nb, NB)], s_sp.at[pl.ds(nb, NB)])
    pltpu.sync_copy(ul2d, u_sp.at[pl.ds(nb, NB)])
    plsc.subcore_barrier()

    _propagate()
    plsc.subcore_barrier()

    pltpu.sync_copy(s_sp.at[pl.ds(nb, NB)], sl2d)

    def _out(i, _):
        r16 = io3 + 2 * i
        sv = plsc.load_gather(sl2d, [r16, io_f])
        uv = plsc.load_gather(ul2d, [r16, io_f])
        dv = plsc.load_gather(dinvl, [r16])
        plsc.store_scatter(sl2d, [r16, io_f], dv * (sv + uv))
        return 0
    lax.fori_loop(0, (NB * H) // L, _out, 0)
    pltpu.sync_copy(sl2d, phH.at[c].at[pl.ds(nb, NB)])


_sc_call = functools.partial(
    pl.kernel,
    out_type=jax.ShapeDtypeStruct((NC, NP, H), jnp.float32),
    mesh=plsc.VectorSubcoreMesh(core_axis_name="c", subcore_axis_name="s",
                                num_cores=NC, num_subcores=NS),
    compiler_params=pltpu.CompilerParams(needs_layout_passes=False,
                                         use_tc_tiling_on_sc=False),
    scratch_types=[
        pltpu.VMEM((CBLK, BLK), jnp.int32),
        pltpu.VMEM((CBLK, BLK), jnp.int32),
        pltpu.VMEM((CBLK * BLK,), jnp.float32),
        pltpu.VMEM((CBLK, BLK, H), jnp.float32),
        pltpu.VMEM((NB, H), jnp.float32),
        pltpu.VMEM((NB, H), jnp.float32),
        pltpu.VMEM((NB,), jnp.float32),
        pltpu.VMEM((NB,), jnp.float32),
        pltpu.VMEM((L,), jnp.float32),
        pltpu.VMEM_SHARED((NP,), jnp.float32),
        pltpu.VMEM_SHARED((NP, H), jnp.float32),
        pltpu.VMEM_SHARED((NP, H), jnp.float32),
        pltpu.SemaphoreType.DMA,
        pltpu.SemaphoreType.DMA,
    ],
)


def kernel(x1, edge_index1, edge_attr1, x2, edge_index2, edge_attr2,
           W1, b1, W2, b2, Wl1, bl1, Wl2, bl2):
    f32 = jnp.float32
    pad_n = EP - E
    pad_idx = (N + (jnp.arange(pad_n, dtype=jnp.int32) % (NP - N)))

    def prep(ei, ea):
        row = jnp.concatenate([ei[0].astype(jnp.int32), pad_idx])
        col = jnp.concatenate([ei[1].astype(jnp.int32), pad_idx])
        ew = jnp.concatenate([ea.astype(f32), jnp.zeros((pad_n,), f32)])
        return row, col, ew

    r1, c1, w1e = prep(edge_index1, edge_attr1)
    r2, c2, w2e = prep(edge_index2, edge_attr2)
    rowH = jnp.stack([r1, r2]).reshape(NC, NBLK, BLK)
    colH = jnp.stack([c1, c2]).reshape(NC, NBLK, BLK)
    ewH = jnp.stack([w1e, w2e])

    xp = jnp.zeros((NC, NP, D), f32).at[:, :N].set(jnp.stack([x1, x2]))
    MB = 2048
    y = pl.pallas_call(
        _mm_body,
        grid=(NC * NP // MB,),
        in_specs=[pl.BlockSpec((MB, D), lambda i: (i, 0)),
                  pl.BlockSpec((D, H), lambda i: (0, 0))],
        out_specs=pl.BlockSpec((MB, H), lambda i: (i, 0)),
        out_shape=jax.ShapeDtypeStruct((NC * NP, H), f32),
    )(xp.reshape(NC * NP, D), W1)
    yH = y.reshape(NC, NP, H)

    zH = jnp.zeros((NP, H), f32)
    b1t = jnp.concatenate([b1, b1]).astype(f32)

    ph = _sc_call(_sc_body)(rowH, colH, ewH, yH, zH, b1t)

    g = pl.pallas_call(
        _readout_body,
        out_shape=jax.ShapeDtypeStruct((NC, H), f32),
        scratch_shapes=[pltpu.VMEM((N, H), jnp.float32)],
    )(ph[:, :N], W2, b2.reshape(1, H))

    out = pl.pallas_call(
        _mlp_body,
        out_shape=jax.ShapeDtypeStruct((1, 1), f32),
    )(g, Wl1, bl1.reshape(1, 4), Wl2, bl2.reshape(1, 1))
    return out.reshape(1)


# speedup vs baseline: 51.8989x; 1.0462x over previous
import functools

import jax
import jax.numpy as jnp
from jax import lax
from jax.experimental import pallas as pl
from jax.experimental.pallas import tpu as pltpu
from jax.experimental.pallas import tpu_sc as plsc

N = 10000
E = 320000
D = 128
H = 8

NC = 2
NS = 16
L = 16

NP = 10240
NB = NP // NS
EP = 327680
BLK = 128
NBLK = EP // BLK
BPT = NBLK // NS
CBLK = 32
NCHUNK = BPT // CBLK


def _leaky(x):
    return jnp.where(x >= 0, x, 0.01 * x)


def _mm_body(x_ref, w_ref, o_ref):
    o_ref[...] = jnp.dot(x_ref[...], w_ref[...],
                         preferred_element_type=jnp.float32)


def _vdot(a, b):
    acc = a[:, 0:1] * b[0:1, :]
    for k in range(1, a.shape[1]):
        acc = acc + a[:, k:k + 1] * b[k:k + 1, :]
    return acc


def _readout_body(ph_ref, w2_ref, b2_ref, g_ref, ones_scr):
    ones_scr[...] = jnp.ones((8, N), jnp.float32)
    for gi in range(NC):
        t = jnp.dot(ph_ref[gi], w2_ref[...],
                    preferred_element_type=jnp.float32) + b2_ref[...]
        t = _leaky(t)
        gsum = jnp.dot(ones_scr[...], t,
                       preferred_element_type=jnp.float32)
        gb = gsum[0:1, :].astype(jnp.bfloat16)
        g_ref[gi:gi + 1, :] = gb.astype(jnp.float32)


def _mlp_body(g_ref, wl1_ref, bl1_ref, wl2_ref, bl2_ref, o_ref):
    z = jnp.concatenate([g_ref[0:1, :], g_ref[1:2, :]], axis=1)
    r = _leaky(jnp.dot(z, wl1_ref[...],
                       preferred_element_type=jnp.float32) + bl1_ref[...])
    o_ref[...] = _vdot(r, wl2_ref[...]) + bl2_ref[...]


def _sc_body(rowH, colH, ewH, yH, zH, b1H, phH,
             row2d, col2d, ewf, urows, ul2d, sl2d, degl, dinvl, b1v,
             deg_sp, u_sp, s_sp, gsem, ssem):
    c = lax.axis_index("c")
    s = lax.axis_index("s")
    nb = s * NB
    bb0 = s * BPT

    iota = lax.iota(jnp.int32, L)
    io3 = jnp.right_shift(iota, 3)
    io_f = jnp.bitwise_and(iota, 7)
    e16m = [16 * i + iota for i in range(BLK // L)]
    f16v = [jnp.full((L,), f, dtype=jnp.int32) for f in range(H)]
    zero16 = jnp.zeros((L,), jnp.float32)

    pltpu.sync_copy(b1H, b1v)

    def _z_deg(i, _):
        degl[pl.ds(i * L, L)] = zero16
        return 0
    lax.fori_loop(0, NB // L, _z_deg, 0)
    pltpu.sync_copy(degl, deg_sp.at[pl.ds(nb, NB)])
    pltpu.sync_copy(zH.at[pl.ds(nb, NB)], s_sp.at[pl.ds(nb, NB)])
    plsc.subcore_barrier()

    def _deg_chunk(k, _):
        bb = bb0 + CBLK * k
        pltpu.sync_copy(colH.at[c].at[pl.ds(bb, CBLK)], col2d)
        pltpu.sync_copy(ewH.at[c].at[pl.ds(bb * BLK, CBLK * BLK)], ewf)

        def _deg_blk(j, _):
            pltpu.async_copy(ewf.at[pl.ds(j * BLK, BLK)],
                             deg_sp.at[col2d.at[j]], ssem, add=True)
            return 0
        lax.fori_loop(0, CBLK, _deg_blk, 0)

        def _deg_drain(j, _):
            pltpu.make_async_copy(ewf.at[pl.ds(0, BLK)],
                                  deg_sp.at[pl.ds(0, BLK)], ssem).wait()
            return 0
        lax.fori_loop(0, CBLK, _deg_drain, 0)
        return 0
    lax.fori_loop(0, NCHUNK, _deg_chunk, 0)
    plsc.subcore_barrier()

    pltpu.sync_copy(deg_sp.at[pl.ds(nb, NB)], degl)

    def _dinv(i, _):
        d = degl[pl.ds(i * L, L)] + 1.0
        half = 0.5 * d
        y = 1.0 / d

        def _newton(_, y):
            return y * (1.5 - half * y * y)
        y = lax.fori_loop(0, 22, _newton, y)
        dinvl[pl.ds(i * L, L)] = y
        return 0
    lax.fori_loop(0, NB // L, _dinv, 0)

    pltpu.sync_copy(yH.at[c].at[pl.ds(nb, NB)], ul2d)

    def _scale_u(i, _):
        r16 = io3 + 2 * i
        u = plsc.load_gather(ul2d, [r16, io_f])
        dv = plsc.load_gather(dinvl, [r16])
        plsc.store_scatter(ul2d, [r16, io_f], u * dv)
        return 0
    lax.fori_loop(0, (NB * H) // L, _scale_u, 0)
    pltpu.sync_copy(ul2d, u_sp.at[pl.ds(nb, NB)])
    plsc.subcore_barrier()

    def _propagate():
        def _chunk(k, _):
            bb = bb0 + CBLK * k
            pltpu.sync_copy(rowH.at[c].at[pl.ds(bb, CBLK)], row2d)
            pltpu.sync_copy(colH.at[c].at[pl.ds(bb, CBLK)], col2d)
            pltpu.sync_copy(ewH.at[c].at[pl.ds(bb * BLK, CBLK * BLK)], ewf)

            def _gfire(j, _):
                pltpu.async_copy(u_sp.at[row2d.at[j]], urows.at[j], gsem)
                return 0
            lax.fori_loop(0, CBLK, _gfire, 0)

            def _gdrain(j, _):
                pltpu.make_async_copy(u_sp.at[pl.ds(0, BLK)],
                                      urows.at[0], gsem).wait()
                return 0
            lax.fori_loop(0, CBLK, _gdrain, 0)

            def _mul_blk(j, _):
                j16 = jnp.full((L,), j, dtype=jnp.int32)
                for i in range(BLK // L):
                    ew16 = ewf[pl.ds(j * BLK + 16 * i, L)]
                    for f in range(H):
                        idx = [j16, e16m[i], f16v[f]]
                        uv = plsc.load_gather(urows, idx)
                        plsc.store_scatter(urows, idx, uv * ew16)
                pltpu.async_copy(urows.at[j], s_sp.at[col2d.at[j]],
                                 ssem, add=True)
                return 0
            lax.fori_loop(0, CBLK, _mul_blk, 0)

            def _sdrain(j, _):
                pltpu.make_async_copy(urows.at[0],
                                      s_sp.at[pl.ds(0, BLK)], ssem).wait()
                return 0
            lax.fori_loop(0, CBLK, _sdrain, 0)
            return 0
        lax.fori_loop(0, NCHUNK, _chunk, 0)

    _propagate()
    plsc.subcore_barrier()

    pltpu.sync_copy(s_sp.at[pl.ds(nb, NB)], sl2d)
    b1vec = b1v[...]

    def _inter(i, _):
        r16 = io3 + 2 * i
        sv = plsc.load_gather(sl2d, [r16, io_f])
        uv = plsc.load_gather(ul2d, [r16, io_f])
        dv = plsc.load_gather(dinvl, [r16])
        z = dv * (sv + uv) + b1vec
        h = jnp.where(z >= 0, z, 0.01 * z)
        plsc.store_scatter(ul2d, [r16, io_f], dv * h)
        return 0
    lax.fori_loop(0, (NB * H) // L, _inter, 0)
    pltpu.sync_copy(zH.at[pl.ds(nb, NB)], s_sp.at[pl.ds(nb, NB)])
    pltpu.sync_copy(ul2d, u_sp.at[pl.ds(nb, NB)])
    plsc.subcore_barrier()

    _propagate()
    plsc.subcore_barrier()

    pltpu.sync_copy(s_sp.at[pl.ds(nb, NB)], sl2d)

    def _out(i, _):
        r16 = io3 + 2 * i
        sv = plsc.load_gather(sl2d, [r16, io_f])
        uv = plsc.load_gather(ul2d, [r16, io_f])
        dv = plsc.load_gather(dinvl, [r16])
        plsc.store_scatter(sl2d, [r16, io_f], dv * (sv + uv))
        return 0
    lax.fori_loop(0, (NB * H) // L, _out, 0)
    pltpu.sync_copy(sl2d, phH.at[c].at[pl.ds(nb, NB)])


_sc_call = functools.partial(
    pl.kernel,
    out_type=jax.ShapeDtypeStruct((NC, NP, H), jnp.float32),
    mesh=plsc.VectorSubcoreMesh(core_axis_name="c", subcore_axis_name="s",
                                num_cores=NC, num_subcores=NS),
    compiler_params=pltpu.CompilerParams(needs_layout_passes=False,
                                         use_tc_tiling_on_sc=False),
    scratch_types=[
        pltpu.VMEM((CBLK, BLK), jnp.int32),
        pltpu.VMEM((CBLK, BLK), jnp.int32),
        pltpu.VMEM((CBLK * BLK,), jnp.float32),
        pltpu.VMEM((CBLK, BLK, H), jnp.float32),
        pltpu.VMEM((NB, H), jnp.float32),
        pltpu.VMEM((NB, H), jnp.float32),
        pltpu.VMEM((NB,), jnp.float32),
        pltpu.VMEM((NB,), jnp.float32),
        pltpu.VMEM((L,), jnp.float32),
        pltpu.VMEM_SHARED((NP,), jnp.float32),
        pltpu.VMEM_SHARED((NP, H), jnp.float32),
        pltpu.VMEM_SHARED((NP, H), jnp.float32),
        pltpu.SemaphoreType.DMA,
        pltpu.SemaphoreType.DMA,
    ],
)


def kernel(x1, edge_index1, edge_attr1, x2, edge_index2, edge_attr2,
           W1, b1, W2, b2, Wl1, bl1, Wl2, bl2):
    f32 = jnp.float32
    pad_n = EP - E
    pad_idx = (N + (jnp.arange(pad_n, dtype=jnp.int32) % (NP - N)))

    def prep(ei, ea):
        row = jnp.concatenate([ei[0].astype(jnp.int32), pad_idx])
        col = jnp.concatenate([ei[1].astype(jnp.int32), pad_idx])
        ew = jnp.concatenate([ea.astype(f32), jnp.zeros((pad_n,), f32)])
        return row, col, ew

    r1, c1, w1e = prep(edge_index1, edge_attr1)
    r2, c2, w2e = prep(edge_index2, edge_attr2)
    rowH = jnp.stack([r1, r2]).reshape(NC, NBLK, BLK)
    colH = jnp.stack([c1, c2]).reshape(NC, NBLK, BLK)
    ewH = jnp.stack([w1e, w2e])

    xp = jnp.zeros((NC, NP, D), f32).at[:, :N].set(jnp.stack([x1, x2]))
    MB = 2048
    y = pl.pallas_call(
        _mm_body,
        grid=(NC * NP // MB,),
        in_specs=[pl.BlockSpec((MB, D), lambda i: (i, 0)),
                  pl.BlockSpec((D, H), lambda i: (0, 0))],
        out_specs=pl.BlockSpec((MB, H), lambda i: (i, 0)),
        out_shape=jax.ShapeDtypeStruct((NC * NP, H), f32),
    )(xp.reshape(NC * NP, D), W1)
    yH = y.reshape(NC, NP, H)

    zH = jnp.zeros((NP, H), f32)
    b1t = jnp.concatenate([b1, b1]).astype(f32)

    ph = _sc_call(_sc_body)(rowH, colH, ewH, yH, zH, b1t)

    g = pl.pallas_call(
        _readout_body,
        out_shape=jax.ShapeDtypeStruct((NC, H), f32),
        scratch_shapes=[pltpu.VMEM((8, N), jnp.float32)],
    )(ph[:, :N], W2, b2.reshape(1, H))

    out = pl.pallas_call(
        _mlp_body,
        out_shape=jax.ShapeDtypeStruct((1, 1), f32),
    )(g, Wl1, bl1.reshape(1, 4), Wl2, bl2.reshape(1, 1))
    return out.reshape(1)
